# baseline (speedup 1.0000x reference)
import jax
import jax.numpy as jnp
from jax import lax
from jax.experimental import pallas as pl
from jax.experimental.pallas import tpu as pltpu
from jax.experimental.pallas import tpu_sc as plsc

_NOISE_ROWS = 64
_ROW = 160000
_NROWS = 2 * _NOISE_ROWS
_NW = 32
_ROWS_PER_W = _NROWS // _NW

_PERM = (
    22, 18, 6, 26, 21, 45, 60, 39, 61, 49, 38, 27, 32, 57, 10, 63,
    35, 20, 24, 56, 52, 40, 51, 42, 55, 4, 31, 14, 0, 43, 34, 3,
    50, 5, 17, 37, 28, 2, 41, 23, 58, 44, 54, 48, 46, 36, 1, 8,
    16, 33, 30, 7, 19, 15, 9, 62, 13, 11, 59, 47, 25, 53, 12, 29,
)

_ROWMAP = _PERM + tuple(range(_NOISE_ROWS, _NROWS))


def _remix_body(src, out, sem):
    cid = lax.axis_index("c")
    sid = lax.axis_index("s")
    wid = sid * 2 + cid
    for w in range(_NW):

        @pl.when(wid == w)
        def _():
            descs = [
                pltpu.async_copy(
                    src.at[_ROWMAP[w * _ROWS_PER_W + i]],
                    out.at[w * _ROWS_PER_W + i],
                    sem,
                )
                for i in range(_ROWS_PER_W)
            ]
            for d in descs:
                d.wait()


def kernel(sources):
    src = sources.reshape(_NROWS, _ROW)
    out = pl.kernel(
        _remix_body,
        out_type=jax.ShapeDtypeStruct((_NROWS, _ROW), jnp.float32),
        mesh=plsc.VectorSubcoreMesh(core_axis_name="c", subcore_axis_name="s"),
        scratch_types=[pltpu.SemaphoreType.DMA],
        name="sc_remix_copy",
    )(src)
    return out.reshape(2, _NOISE_ROWS, 1, _ROW)


# speedup vs baseline: 34.0243x; 34.0243x over previous
import jax
import jax.numpy as jnp
from jax import lax
from jax.experimental import pallas as pl
from jax.experimental.pallas import tpu as pltpu
from jax.experimental.pallas import tpu_sc as plsc

_NOISE_ROWS = 64
_ROW = 160000
_NROWS = 2 * _NOISE_ROWS
_NC = 2
_NS = 16
_NW = _NC * _NS
_ROWS_PER_W = _NROWS // _NW
_CHUNK = 40000
_CHUNKS_PER_ROW = _ROW // _CHUNK
_STEPS = _ROWS_PER_W * _CHUNKS_PER_ROW

_PERM = (
    22, 18, 6, 26, 21, 45, 60, 39, 61, 49, 38, 27, 32, 57, 10, 63,
    35, 20, 24, 56, 52, 40, 51, 42, 55, 4, 31, 14, 0, 43, 34, 3,
    50, 5, 17, 37, 28, 2, 41, 23, 58, 44, 54, 48, 46, 36, 1, 8,
    16, 33, 30, 7, 19, 15, 9, 62, 13, 11, 59, 47, 25, 53, 12, 29,
)

_ROWMAP = _PERM + tuple(range(_NOISE_ROWS, _NROWS))


def _remix_body(src, out, buf, sem_in, sem_out):
    cid = lax.axis_index("c")
    sid = lax.axis_index("s")
    wid = sid * _NC + cid

    srows = []
    for i in range(_ROWS_PER_W):
        m = jnp.int32(_ROWMAP[i])
        for w in range(1, _NW):
            m = jnp.where(wid == w, _ROWMAP[w * _ROWS_PER_W + i], m)
        srows.append(m)
    orow0 = wid * _ROWS_PER_W

    def in_copy(t, slot):
        i, c = divmod(t, _CHUNKS_PER_ROW)
        off = pl.multiple_of(srows[i] * _ROW + c * _CHUNK, 8)
        return pltpu.async_copy(
            src.at[pl.ds(off, _CHUNK)], buf.at[pl.ds(slot * _CHUNK, _CHUNK)], sem_in
        )

    def out_copy(t, slot):
        i, c = divmod(t, _CHUNKS_PER_ROW)
        off = pl.multiple_of((orow0 + i) * _ROW + c * _CHUNK, 8)
        return pltpu.async_copy(
            buf.at[pl.ds(slot * _CHUNK, _CHUNK)], out.at[pl.ds(off, _CHUNK)], sem_out
        )

    ins, outs = {}, {}
    ins[0] = in_copy(0, 0)
    for t in range(_STEPS):
        slot = t % 2
        ins[t].wait()
        if t + 1 < _STEPS:
            if t - 1 >= 0:
                outs[t - 1].wait()
            ins[t + 1] = in_copy(t + 1, (t + 1) % 2)
        outs[t] = out_copy(t, slot)
    outs[_STEPS - 2].wait()
    outs[_STEPS - 1].wait()


def kernel(sources):
    src = sources.reshape(_NROWS * _ROW)
    out = pl.kernel(
        _remix_body,
        out_type=jax.ShapeDtypeStruct((_NROWS * _ROW,), jnp.float32),
        mesh=plsc.VectorSubcoreMesh(core_axis_name="c", subcore_axis_name="s"),
        scratch_types=[
            pltpu.VMEM((2 * _CHUNK,), jnp.float32),
            pltpu.SemaphoreType.DMA,
            pltpu.SemaphoreType.DMA,
        ],
        name="sc_remix_copy",
    )(src)
    return out.reshape(2, _NOISE_ROWS, 1, _ROW)


# speedup vs baseline: 35.2597x; 1.0363x over previous
import jax
import jax.numpy as jnp
from jax import lax
from jax.experimental import pallas as pl
from jax.experimental.pallas import tpu as pltpu
from jax.experimental.pallas import tpu_sc as plsc

_NOISE_ROWS = 64
_ROW = 160000
_NROWS = 2 * _NOISE_ROWS
_NC = 2
_NS = 16
_NW = _NC * _NS
_ROWS_PER_W = _NROWS // _NW
_CHUNK = 40000
_CHUNKS_PER_ROW = _ROW // _CHUNK
_STEPS = _ROWS_PER_W * _CHUNKS_PER_ROW
_NBUF = 3

_PERM = (
    22, 18, 6, 26, 21, 45, 60, 39, 61, 49, 38, 27, 32, 57, 10, 63,
    35, 20, 24, 56, 52, 40, 51, 42, 55, 4, 31, 14, 0, 43, 34, 3,
    50, 5, 17, 37, 28, 2, 41, 23, 58, 44, 54, 48, 46, 36, 1, 8,
    16, 33, 30, 7, 19, 15, 9, 62, 13, 11, 59, 47, 25, 53, 12, 29,
)

_ROWMAP = _PERM + tuple(range(_NOISE_ROWS, _NROWS))


def _remix_body(src, out, buf, sem_in, sem_out):
    cid = lax.axis_index("c")
    sid = lax.axis_index("s")
    wid = sid * _NC + cid

    srows = []
    for i in range(_ROWS_PER_W):
        m = jnp.int32(_ROWMAP[i])
        for w in range(1, _NW):
            m = jnp.where(wid == w, _ROWMAP[w * _ROWS_PER_W + i], m)
        srows.append(m)
    orow0 = wid * _ROWS_PER_W

    def in_copy(t, slot):
        i, c = divmod(t, _CHUNKS_PER_ROW)
        off = pl.multiple_of(srows[i] * _ROW + c * _CHUNK, 8)
        return pltpu.async_copy(
            src.at[pl.ds(off, _CHUNK)], buf.at[pl.ds(slot * _CHUNK, _CHUNK)], sem_in
        )

    def out_copy(t, slot):
        i, c = divmod(t, _CHUNKS_PER_ROW)
        off = pl.multiple_of((orow0 + i) * _ROW + c * _CHUNK, 8)
        return pltpu.async_copy(
            buf.at[pl.ds(slot * _CHUNK, _CHUNK)], out.at[pl.ds(off, _CHUNK)], sem_out
        )

    ins, outs = {}, {}
    ins[0] = in_copy(0, 0)
    ins[1] = in_copy(1, 1)
    for t in range(_STEPS):
        ins[t].wait()
        outs[t] = out_copy(t, t % _NBUF)
        if t + 2 < _STEPS:
            if t - 1 >= 0:
                outs[t - 1].wait()
            ins[t + 2] = in_copy(t + 2, (t + 2) % _NBUF)
    for t in range(_STEPS - 3, _STEPS):
        outs[t].wait()


def kernel(sources):
    src = sources.reshape(_NROWS * _ROW)
    out = pl.kernel(
        _remix_body,
        out_type=jax.ShapeDtypeStruct((_NROWS * _ROW,), jnp.float32),
        mesh=plsc.VectorSubcoreMesh(core_axis_name="c", subcore_axis_name="s"),
        scratch_types=[
            pltpu.VMEM((_NBUF * _CHUNK,), jnp.float32),
            pltpu.SemaphoreType.DMA,
            pltpu.SemaphoreType.DMA,
        ],
        name="sc_remix_copy",
    )(src)
    return out.reshape(2, _NOISE_ROWS, 1, _ROW)
